# baseline (speedup 1.0000x reference)
import functools

import jax
import jax.numpy as jnp
from jax.experimental import pallas as pl
from jax.experimental.pallas import tpu as pltpu

B, L, D, D_OUT = 16, 2048, 1024, 1024
TL = 512
NT = L // TL


def _body(x_ref, mask_ref, w1_ref, v_ref, wr_ref, br_ref, out_ref,
          acc_ref, m_ref, s_ref):
    j = pl.program_id(1)

    @pl.when(j == 0)
    def _init():
        m_ref[0, 0] = jnp.float32(-1e30)
        s_ref[0, 0] = jnp.float32(0.0)
        acc_ref[...] = jnp.zeros_like(acc_ref)

    x = x_ref[0]
    xb = x.astype(jnp.bfloat16)
    h = jnp.tanh(
        jax.lax.dot_general(xb, w1_ref[...], (((1,), (0,)), ((), ())),
                            preferred_element_type=jnp.float32))
    scores = jax.lax.dot_general(v_ref[...], h, (((1,), (1,)), ((), ())),
                                 preferred_element_type=jnp.float32)
    mask = mask_ref[0, 0]
    scores = jnp.where(mask > 0, scores, jnp.float32(-1e9))

    m_old = m_ref[0, 0]
    m_new = jnp.maximum(m_old, jnp.max(scores))
    alpha = jnp.exp(m_old - m_new)
    p = jnp.exp(scores - m_new)
    s_ref[0, 0] = s_ref[0, 0] * alpha + jnp.sum(p)
    acc_ref[...] = acc_ref[...] * alpha + jax.lax.dot_general(
        p, x, (((1,), (0,)), ((), ())), preferred_element_type=jnp.float32)
    m_ref[0, 0] = m_new

    @pl.when(j == NT - 1)
    def _finish():
        pooled = acc_ref[...] / s_ref[0, 0]
        out_ref[0] = jax.lax.dot_general(
            pooled, wr_ref[...], (((1,), (0,)), ((), ())),
            preferred_element_type=jnp.float32) + br_ref[...]


@functools.partial(jax.jit, static_argnames=())
def kernel(word_hidden, word_mask, W1, v, Wr, br):
    maskf = word_mask.astype(jnp.float32).reshape(B, NT, 1, TL)
    w1_bf = W1.astype(jnp.bfloat16)
    v2 = v.reshape(1, D)
    br2 = br.reshape(1, D_OUT)
    out = pl.pallas_call(
        _body,
        grid=(B, NT),
        in_specs=[
            pl.BlockSpec((1, TL, D), lambda b, j: (b, j, 0)),
            pl.BlockSpec((1, 1, 1, TL), lambda b, j: (b, j, 0, 0)),
            pl.BlockSpec((D, D), lambda b, j: (0, 0)),
            pl.BlockSpec((1, D), lambda b, j: (0, 0)),
            pl.BlockSpec((D, D_OUT), lambda b, j: (0, 0)),
            pl.BlockSpec((1, D_OUT), lambda b, j: (0, 0)),
        ],
        out_specs=pl.BlockSpec((1, 1, D_OUT), lambda b, j: (b, 0, 0)),
        out_shape=jax.ShapeDtypeStruct((B, 1, D_OUT), jnp.float32),
        scratch_shapes=[
            pltpu.VMEM((1, D), jnp.float32),
            pltpu.SMEM((1, 1), jnp.float32),
            pltpu.SMEM((1, 1), jnp.float32),
        ],
        compiler_params=pltpu.CompilerParams(
            dimension_semantics=("parallel", "arbitrary")),
    )(word_hidden, maskf, w1_bf, v2, Wr, br2)
    return out.reshape(B, D_OUT)


# speedup vs baseline: 1.1349x; 1.1349x over previous
import functools

import jax
import jax.numpy as jnp
from jax.experimental import pallas as pl
from jax.experimental.pallas import tpu as pltpu

B, L, D, D_OUT = 16, 2048, 1024, 1024


def _body(x_ref, mask_ref, w1_ref, v_ref, wr_ref, br_ref, out_ref):
    x = x_ref[0]
    xb = x.astype(jnp.bfloat16)
    h = jnp.tanh(
        jax.lax.dot_general(xb, w1_ref[...], (((1,), (0,)), ((), ())),
                            preferred_element_type=jnp.float32))
    scores = jax.lax.dot_general(v_ref[...], h, (((1,), (1,)), ((), ())),
                                 preferred_element_type=jnp.float32)
    scores = jnp.where(mask_ref[0] > 0, scores, jnp.float32(-1e9))
    m = jnp.max(scores)
    p = jnp.exp(scores - m)
    s = jnp.sum(p)
    pooled = jax.lax.dot_general(p, x, (((1,), (0,)), ((), ())),
                                 preferred_element_type=jnp.float32) / s
    out_ref[0] = jax.lax.dot_general(
        pooled, wr_ref[...], (((1,), (0,)), ((), ())),
        preferred_element_type=jnp.float32) + br_ref[...]


@functools.partial(jax.jit, static_argnames=())
def kernel(word_hidden, word_mask, W1, v, Wr, br):
    maskf = word_mask.astype(jnp.float32).reshape(B, 1, L)
    w1_bf = W1.astype(jnp.bfloat16)
    v2 = v.reshape(1, D)
    br2 = br.reshape(1, D_OUT)
    out = pl.pallas_call(
        _body,
        grid=(B,),
        in_specs=[
            pl.BlockSpec((1, L, D), lambda b: (b, 0, 0)),
            pl.BlockSpec((1, 1, L), lambda b: (b, 0, 0)),
            pl.BlockSpec((D, D), lambda b: (0, 0)),
            pl.BlockSpec((1, D), lambda b: (0, 0)),
            pl.BlockSpec((D, D_OUT), lambda b: (0, 0)),
            pl.BlockSpec((1, D_OUT), lambda b: (0, 0)),
        ],
        out_specs=pl.BlockSpec((1, 1, D_OUT), lambda b: (b, 0, 0)),
        out_shape=jax.ShapeDtypeStruct((B, 1, D_OUT), jnp.float32),
        compiler_params=pltpu.CompilerParams(
            dimension_semantics=("arbitrary",)),
    )(word_hidden, maskf, w1_bf, v2, Wr, br2)
    return out.reshape(B, D_OUT)
